# Initial kernel scaffold; baseline (speedup 1.0000x reference)
import jax
import jax.numpy as jnp
from jax.experimental import pallas as pl


def kernel(x, edge_index, batch, bn_in_g, bn_in_b, Wn0, Ws0, bc0, bn_g0, bn_b0, Wn1, Ws1, bc1, bn_g1, bn_b1, Wn2, Ws2, bc2, bn_g2, bn_b2, Wh0, bh0, Wh1, bh1):
    raise NotImplementedError("write your pallas kernel here")


# speedup vs baseline: 6.7875x; 6.7875x over previous
import functools

import jax
import jax.numpy as jnp
from jax import lax
from jax.experimental import pallas as pl
from jax.experimental.pallas import tpu as pltpu
from jax.experimental.pallas import tpu_sc as plsc

N = 50000
E = 800000
G = 1000
HID = 64
EMB = 32

NC = 2
NS = 16

_MESH = dict(core_axis_name="c", subcore_axis_name="s", num_cores=NC,
             num_subcores=NS)
_SC_PARAMS = dict(
    mesh=plsc.VectorSubcoreMesh(**_MESH),
    compiler_params=pltpu.CompilerParams(use_tc_tiling_on_sc=False),
)


def _dot_t(a, w):
    return lax.dot_general(a, w, (((1,), (1,)), ((), ())),
                           preferred_element_type=jnp.float32)


def _fill_vmem_rows(ref, nrows, ncols, val):
    v16 = jnp.full((16,), val, jnp.float32)

    def body(i, _):
        for j in range(ncols // 16):
            ref[i, pl.ds(j * 16, 16)] = v16
        return 0

    lax.fori_loop(0, nrows, body, 0, unroll=False)


_CP = 3128
_CPLAST = N - (NS - 1) * _CP


def _per_tile_rows(s, fn):
    @pl.when(s < NS - 1)
    def _():
        fn(s * _CP, _CP)

    @pl.when(s == NS - 1)
    def _():
        fn((NS - 1) * _CP, _CPLAST)


def _copy_zero_slice(zbuf, acc, base, total, chunk):
    nfull = total // chunk
    rem = total - nfull * chunk
    for k in range(nfull):
        pltpu.sync_copy(zbuf, acc.at[pl.ds(base + k * chunk, chunk)])
    if rem:
        pltpu.sync_copy(zbuf.at[pl.ds(0, rem)],
                        acc.at[pl.ds(base + nfull * chunk, rem)])


def _sc_agg16(h0p, src, dst):
    EPW = E // (NC * NS)
    C = 1000
    NIT = EPW // C

    @functools.partial(
        pl.kernel,
        out_type=jax.ShapeDtypeStruct((NC, N, 16), jnp.float32),
        scratch_types=[
            pltpu.VMEM((C,), jnp.int32),
            pltpu.VMEM((C,), jnp.int32),
            pltpu.VMEM((C, 16), jnp.float32),
            pltpu.VMEM_SHARED((N, 16), jnp.float32),
        ],
        **_SC_PARAMS,
    )
    def k(h_hbm, src_hbm, dst_hbm, out_hbm, sidx, didx, rows, acc):
        c = lax.axis_index("c")
        s = lax.axis_index("s")
        w = s * NC + c
        _fill_vmem_rows(rows, C, 16, 0.0)
        _per_tile_rows(s, lambda base, n: _copy_zero_slice(rows, acc, base, n, C))
        plsc.subcore_barrier()

        base_e = w * EPW

        def body(i, _):
            off = base_e + i * C
            pltpu.sync_copy(src_hbm.at[pl.ds(off, C)], sidx)
            pltpu.sync_copy(dst_hbm.at[pl.ds(off, C)], didx)
            pltpu.sync_copy(h_hbm.at[sidx], rows)
            pltpu.sync_copy(rows, acc.at[didx], add=True)
            return 0

        lax.fori_loop(0, NIT, body, 0, unroll=False)
        plsc.subcore_barrier()
        _per_tile_rows(s, lambda base, n: pltpu.sync_copy(
            acc.at[pl.ds(base, n)], out_hbm.at[c, pl.ds(base, n)]))

    return k(h0p, src, dst)


def _sc_agg_split(h0, h1, h2, h3, src, dst):
    EPT = E // NS
    C = 1000
    NIT = EPT // C

    @functools.partial(
        pl.kernel,
        out_type=jax.ShapeDtypeStruct((4, N, 16), jnp.float32),
        scratch_types=[
            pltpu.VMEM((C,), jnp.int32),
            pltpu.VMEM((C,), jnp.int32),
            pltpu.VMEM((C, 16), jnp.float32),
            pltpu.VMEM_SHARED((N, 16), jnp.float32),
        ],
        **_SC_PARAMS,
    )
    def k(h0_hbm, h1_hbm, h2_hbm, h3_hbm, src_hbm, dst_hbm, out_hbm,
          sidx, didx, rows, acc):
        c = lax.axis_index("c")
        s = lax.axis_index("s")
        base_e = s * EPT

        for p in range(2):
            _fill_vmem_rows(rows, C, 16, 0.0)
            _per_tile_rows(s, lambda base, n: _copy_zero_slice(
                rows, acc, base, n, C))
            plsc.subcore_barrier()

            def body(i, _, p=p):
                off = base_e + i * C
                pltpu.sync_copy(src_hbm.at[pl.ds(off, C)], sidx)
                pltpu.sync_copy(dst_hbm.at[pl.ds(off, C)], didx)

                @pl.when(c == 0)
                def _():
                    tab = h0_hbm if p == 0 else h1_hbm
                    pltpu.sync_copy(tab.at[sidx], rows)

                @pl.when(c == 1)
                def _():
                    tab = h2_hbm if p == 0 else h3_hbm
                    pltpu.sync_copy(tab.at[sidx], rows)

                pltpu.sync_copy(rows, acc.at[didx], add=True)
                return 0

            lax.fori_loop(0, NIT, body, 0, unroll=False)
            plsc.subcore_barrier()
            _per_tile_rows(s, lambda base, n: pltpu.sync_copy(
                acc.at[pl.ds(base, n)], out_hbm.at[c * 2 + p, pl.ds(base, n)]))
            if p == 0:
                plsc.subcore_barrier()

    return k(h0, h1, h2, h3, src, dst)


def _sc_pool(h0, h1, h2, h3, batch):
    CH = 400
    NCHUNK = N // CH
    JMAX = (NCHUNK + NS - 1) // NS

    @functools.partial(
        pl.kernel,
        out_type=(jax.ShapeDtypeStruct((4, G, 16), jnp.float32),
                  jax.ShapeDtypeStruct((G, 16), jnp.float32)),
        scratch_types=[
            pltpu.VMEM((CH,), jnp.int32),
            pltpu.VMEM((CH, 16), jnp.float32),
            pltpu.VMEM((CH, 16), jnp.float32),
            pltpu.VMEM((CH, 16), jnp.float32),
            pltpu.VMEM_SHARED((G, 16), jnp.float32),
            pltpu.VMEM_SHARED((G, 16), jnp.float32),
            pltpu.VMEM_SHARED((G, 16), jnp.float32),
        ],
        **_SC_PARAMS,
    )
    def k(h0_hbm, h1_hbm, h2_hbm, h3_hbm, batch_hbm, outp_hbm, outc_hbm,
          bidx, rows_a, rows_b, ones_v, acc_a, acc_b, acc_c):
        c = lax.axis_index("c")
        s = lax.axis_index("s")
        _fill_vmem_rows(ones_v, CH, 16, 1.0)
        _fill_vmem_rows(rows_a, CH, 16, 0.0)

        @pl.when(s == 0)
        def _():
            _copy_zero_slice(rows_a, acc_a, 0, G, CH)

        @pl.when(s == 1)
        def _():
            _copy_zero_slice(rows_a, acc_b, 0, G, CH)

        @pl.when(s == 2)
        def _():
            _copy_zero_slice(rows_a, acc_c, 0, G, CH)

        plsc.subcore_barrier()

        def body(j, _):
            ch = s + NS * j

            @pl.when(ch < NCHUNK)
            def _():
                off = ch * CH
                pltpu.sync_copy(batch_hbm.at[pl.ds(off, CH)], bidx)

                @pl.when(c == 0)
                def _():
                    pltpu.sync_copy(h0_hbm.at[pl.ds(off, CH)], rows_a)
                    pltpu.sync_copy(h1_hbm.at[pl.ds(off, CH)], rows_b)
                    pltpu.sync_copy(ones_v, acc_c.at[bidx], add=True)

                @pl.when(c == 1)
                def _():
                    pltpu.sync_copy(h2_hbm.at[pl.ds(off, CH)], rows_a)
                    pltpu.sync_copy(h3_hbm.at[pl.ds(off, CH)], rows_b)

                pltpu.sync_copy(rows_a, acc_a.at[bidx], add=True)
                pltpu.sync_copy(rows_b, acc_b.at[bidx], add=True)

            return 0

        lax.fori_loop(0, JMAX, body, 0, unroll=False)
        plsc.subcore_barrier()

        @pl.when(s == 0)
        def _():
            pltpu.sync_copy(acc_a, outp_hbm.at[c * 2])

        @pl.when(s == 1)
        def _():
            pltpu.sync_copy(acc_b, outp_hbm.at[c * 2 + 1])

        @pl.when((s == 2) & (c == 0))
        def _():
            pltpu.sync_copy(acc_c, outc_hbm)

    return k(h0, h1, h2, h3, batch)


_BLK = 1000
_NB = N // _BLK


def _tc_input_bn(x, g, b):
    def stats_body(x_ref, st_ref):
        i = pl.program_id(0)
        xv = x_ref[...]
        _accum_stats(i, xv, st_ref)

    st = pl.pallas_call(
        stats_body, grid=(_NB,),
        in_specs=[pl.BlockSpec((_BLK, 4), lambda i: (i, 0))],
        out_specs=pl.BlockSpec((2, 4), lambda i: (0, 0)),
        out_shape=jax.ShapeDtypeStruct((2, 4), jnp.float32))(x)

    def norm_body(x_ref, st_ref, g_ref, b_ref, o_ref):
        m = st_ref[0] / N
        v = st_ref[1] / N - m * m
        h = (x_ref[...] - m) * lax.rsqrt(v + 1e-5) * g_ref[...] + b_ref[...]
        o_ref[...] = jnp.concatenate(
            [h, jnp.ones((_BLK, 1), jnp.float32),
             jnp.zeros((_BLK, 11), jnp.float32)], axis=1)

    return pl.pallas_call(
        norm_body, grid=(_NB,),
        in_specs=[
            pl.BlockSpec((_BLK, 4), lambda i: (i, 0)),
            pl.BlockSpec((2, 4), lambda i: (0, 0)),
            pl.BlockSpec((4,), lambda i: (0,)),
            pl.BlockSpec((4,), lambda i: (0,)),
        ],
        out_specs=pl.BlockSpec((_BLK, 16), lambda i: (i, 0)),
        out_shape=jax.ShapeDtypeStruct((N, 16), jnp.float32))(x, st, g, b)


def _accum_stats(i, z, st_ref):
    st = jnp.concatenate([jnp.sum(z, axis=0)[None, :],
                          jnp.sum(z * z, axis=0)[None, :]], axis=0)

    @pl.when(i == 0)
    def _():
        st_ref[...] = st

    @pl.when(i > 0)
    def _():
        st_ref[...] = st_ref[...] + st


def _tc_layer0_z(part, h0p, Wn, Ws, bc):
    def body(p_ref, h_ref, wn_ref, ws_ref, bc_ref, z_ref, st_ref, dg_ref):
        i = pl.program_id(0)
        psum = p_ref[0] + p_ref[1]
        deginv = 1.0 / jnp.maximum(psum[:, 4], 1.0)
        dg_ref[...] = deginv[None, None, :]
        agg = psum[:, :4] * deginv[:, None]
        h0 = h_ref[...][:, :4]
        z = _dot_t(agg, wn_ref[...]) + _dot_t(h0, ws_ref[...]) + bc_ref[...]
        z_ref[...] = z
        _accum_stats(i, z, st_ref)

    return pl.pallas_call(
        body, grid=(_NB,),
        in_specs=[
            pl.BlockSpec((NC, _BLK, 16), lambda i: (0, i, 0)),
            pl.BlockSpec((_BLK, 16), lambda i: (i, 0)),
            pl.BlockSpec((HID, 4), lambda i: (0, 0)),
            pl.BlockSpec((HID, 4), lambda i: (0, 0)),
            pl.BlockSpec((HID,), lambda i: (0,)),
        ],
        out_specs=[
            pl.BlockSpec((_BLK, HID), lambda i: (i, 0)),
            pl.BlockSpec((2, HID), lambda i: (0, 0)),
            pl.BlockSpec((1, 1, _BLK), lambda i: (i, 0, 0)),
        ],
        out_shape=[
            jax.ShapeDtypeStruct((N, HID), jnp.float32),
            jax.ShapeDtypeStruct((2, HID), jnp.float32),
            jax.ShapeDtypeStruct((_NB, 1, _BLK), jnp.float32),
        ])(part, h0p, Wn, Ws, bc)


def _tc_layer_z(sums, deginv, hs, Wn, Ws, bc):
    def body(su_ref, dg_ref, h0_ref, h1_ref, h2_ref, h3_ref,
             wn_ref, ws_ref, bc_ref, z_ref, st_ref):
        i = pl.program_id(0)
        s = jnp.concatenate([su_ref[0], su_ref[1], su_ref[2], su_ref[3]],
                            axis=1)
        agg = s * dg_ref[0, 0][:, None]
        h = jnp.concatenate([h0_ref[...], h1_ref[...], h2_ref[...],
                             h3_ref[...]], axis=1)
        z = _dot_t(agg, wn_ref[...]) + _dot_t(h, ws_ref[...]) + bc_ref[...]
        z_ref[...] = z
        _accum_stats(i, z, st_ref)

    return pl.pallas_call(
        body, grid=(_NB,),
        in_specs=[
            pl.BlockSpec((4, _BLK, 16), lambda i: (0, i, 0)),
            pl.BlockSpec((1, 1, _BLK), lambda i: (i, 0, 0)),
        ] + [pl.BlockSpec((_BLK, 16), lambda i: (i, 0))] * 4 + [
            pl.BlockSpec((HID, HID), lambda i: (0, 0)),
            pl.BlockSpec((HID, HID), lambda i: (0, 0)),
            pl.BlockSpec((HID,), lambda i: (0,)),
        ],
        out_specs=[
            pl.BlockSpec((_BLK, HID), lambda i: (i, 0)),
            pl.BlockSpec((2, HID), lambda i: (0, 0)),
        ],
        out_shape=[
            jax.ShapeDtypeStruct((N, HID), jnp.float32),
            jax.ShapeDtypeStruct((2, HID), jnp.float32),
        ])(sums, deginv, *hs, Wn, Ws, bc)


def _tc_bnrelu(z, st, g, b):
    def body(z_ref, st_ref, g_ref, b_ref, h0_ref, h1_ref, h2_ref, h3_ref):
        m = st_ref[0] / N
        v = st_ref[1] / N - m * m
        y = (z_ref[...] - m) * lax.rsqrt(v + 1e-5) * g_ref[...] + b_ref[...]
        y = jnp.maximum(y, 0.0)
        h0_ref[...] = y[:, 0:16]
        h1_ref[...] = y[:, 16:32]
        h2_ref[...] = y[:, 32:48]
        h3_ref[...] = y[:, 48:64]

    return pl.pallas_call(
        body, grid=(_NB,),
        in_specs=[
            pl.BlockSpec((_BLK, HID), lambda i: (i, 0)),
            pl.BlockSpec((2, HID), lambda i: (0, 0)),
            pl.BlockSpec((HID,), lambda i: (0,)),
            pl.BlockSpec((HID,), lambda i: (0,)),
        ],
        out_specs=[pl.BlockSpec((_BLK, 16), lambda i: (i, 0))] * 4,
        out_shape=[jax.ShapeDtypeStruct((N, 16), jnp.float32)] * 4,
    )(z, st, g, b)


def _tc_readout(poolp, poolc, Wh0, bh0, Wh1, bh1):
    def body(p_ref, c_ref, w0_ref, b0_ref, w1_ref, b1_ref, o_ref):
        pooled = jnp.concatenate([p_ref[0], p_ref[1], p_ref[2], p_ref[3]],
                                 axis=1)
        cnt = jnp.maximum(c_ref[...][:, 0], 1.0)
        mean = pooled / cnt[:, None]
        z = jnp.maximum(_dot_t(mean, w0_ref[...]) + b0_ref[...], 0.0)
        z2 = _dot_t(z, w1_ref[...]) + b1_ref[...]
        n = jnp.sqrt(jnp.sum(z2 * z2, axis=1, keepdims=True))
        o_ref[...] = z2 / jnp.maximum(n, 1e-12)

    return pl.pallas_call(
        body,
        out_shape=jax.ShapeDtypeStruct((G, EMB), jnp.float32))(
            poolp, poolc, Wh0, bh0, Wh1, bh1)


def kernel(x, edge_index, batch, bn_in_g, bn_in_b,
           Wn0, Ws0, bc0, bn_g0, bn_b0,
           Wn1, Ws1, bc1, bn_g1, bn_b1,
           Wn2, Ws2, bc2, bn_g2, bn_b2,
           Wh0, bh0, Wh1, bh1):
    src = edge_index[0]
    dst = edge_index[1]

    h0p = _tc_input_bn(x, bn_in_g, bn_in_b)

    part0 = _sc_agg16(h0p, src, dst)
    z0, st0, deginv = _tc_layer0_z(part0, h0p, Wn0, Ws0, bc0)
    h1 = _tc_bnrelu(z0, st0, bn_g0, bn_b0)

    sum1 = _sc_agg_split(*h1, src, dst)
    z1, st1 = _tc_layer_z(sum1, deginv, h1, Wn1, Ws1, bc1)
    h2 = _tc_bnrelu(z1, st1, bn_g1, bn_b1)

    sum2 = _sc_agg_split(*h2, src, dst)
    z2, st2 = _tc_layer_z(sum2, deginv, h2, Wn2, Ws2, bc2)
    h3 = _tc_bnrelu(z2, st2, bn_g2, bn_b2)

    poolp, poolc = _sc_pool(*h3, batch)
    return _tc_readout(poolp, poolc, Wh0, bh0, Wh1, bh1)
